# baseline (speedup 1.0000x reference)
import jax
import jax.numpy as jnp
import numpy as np
from jax.experimental import pallas as pl
from jax.experimental.pallas import tpu as pltpu

_SHAPE = (128, 32768)
_N = _SHAPE[0] * _SHAPE[1]
_LOW_RANK = 4026530
_LOW_W = np.float32(0.25)
_HIGH_W = np.float32(0.75)
_MASK31 = np.int32(0x7FFFFFFF)
_INT_MIN = np.int32(-(2**31))
_INT_MAX = np.int32(2**31 - 1)

_ROWS_PER_BLK = 8
_NBLK = _SHAPE[0] // _ROWS_PER_BLK


def _key_to_f32(k):
    b = k ^ (jax.lax.shift_right_arithmetic(k, 31) & _MASK31)
    return jax.lax.bitcast_convert_type(b, jnp.float32)


def _body(x_ref, o_ref, scr_ref, tm_ref):
    i = pl.program_id(0)

    @pl.when(i < _NBLK)
    def _load():
        x = x_ref[...]
        b = jax.lax.bitcast_convert_type(x, jnp.int32)
        keys = b ^ (jax.lax.shift_right_arithmetic(b, 31) & _MASK31)
        scr_ref[pl.ds(i * _ROWS_PER_BLK, _ROWS_PER_BLK), :] = keys

    @pl.when(i == _NBLK)
    def _search():
        def count_lt(q):
            def chunk(j, acc):
                c = scr_ref[pl.ds(j * _ROWS_PER_BLK, _ROWS_PER_BLK), :]
                m = (c < q).astype(jnp.int32)
                return acc + m.reshape(8, 64, 4, 128).sum(axis=1)
            acc = jax.lax.fori_loop(
                0, _NBLK, chunk, jnp.zeros((8, 4, 128), jnp.int32),
                unroll=4)
            return jnp.sum(acc)

        def step(s, p):
            bit = jnp.left_shift(np.int32(1), (31 - s).astype(jnp.int32))
            q = p + bit
            c = count_lt(q)
            return jnp.where(c <= _LOW_RANK, q, p)

        p = jax.lax.fori_loop(0, 32, step, _INT_MIN)

        def succ_chunk(j, carry):
            c_le, mn_above = carry
            c = scr_ref[pl.ds(j * _ROWS_PER_BLK, _ROWS_PER_BLK), :]
            c_le = c_le + jnp.sum((c <= p).astype(jnp.int32))
            above = jnp.where(c > p, c, _INT_MAX)
            return c_le, jnp.minimum(mn_above, jnp.min(above))

        c_le, mn_above = jax.lax.fori_loop(
            0, _NBLK, succ_chunk, (jnp.int32(0), _INT_MAX))
        p_high = jnp.where(c_le >= _LOW_RANK + 2, p, mn_above)

        v_low = _key_to_f32(p)
        v_high = _key_to_f32(p_high)
        tresh = v_low * _LOW_W + v_high * _HIGH_W
        tm_ref[0] = tresh
        tm_ref[1] = jnp.where(tresh >= v_high, v_high, v_low)

    @pl.when(i == _NBLK + 1)
    def _apply():
        tresh = tm_ref[0]
        m = tm_ref[1]

        def chunk(j, carry):
            keys = scr_ref[pl.ds(j * _ROWS_PER_BLK, _ROWS_PER_BLK), :]
            x = _key_to_f32(keys)
            o_ref[pl.ds(j * _ROWS_PER_BLK, _ROWS_PER_BLK), :] = (
                jnp.where(x > tresh, m, x))
            return carry

        jax.lax.fori_loop(0, _NBLK, chunk, jnp.int32(0))


@jax.jit
def kernel(tensor):
    return pl.pallas_call(
        _body,
        grid=(_NBLK + 2,),
        in_specs=[pl.BlockSpec(
            (_ROWS_PER_BLK, _SHAPE[1]),
            lambda i: (jnp.minimum(i, _NBLK - 1), 0))],
        out_specs=pl.BlockSpec(_SHAPE, lambda i: (0, 0)),
        out_shape=jax.ShapeDtypeStruct(_SHAPE, jnp.float32),
        scratch_shapes=[pltpu.VMEM(_SHAPE, jnp.int32),
                        pltpu.SMEM((2,), jnp.float32)],
    )(tensor)


# speedup vs baseline: 1.6054x; 1.6054x over previous
import jax
import jax.numpy as jnp
import numpy as np
from jax.experimental import pallas as pl
from jax.experimental.pallas import tpu as pltpu

_SHAPE = (128, 32768)
_N = _SHAPE[0] * _SHAPE[1]
_LOW_RANK = 4026530
_LOW_W = np.float32(0.25)
_HIGH_W = np.float32(0.75)
_MASK31 = np.int32(0x7FFFFFFF)
_INT_MIN = np.int32(-(2**31))
_INT_MAX = np.int32(2**31 - 1)

_ROWS_PER_BLK = 8
_NBLK = _SHAPE[0] // _ROWS_PER_BLK


def _key_to_f32(k):
    b = k ^ (jax.lax.shift_right_arithmetic(k, 31) & _MASK31)
    return jax.lax.bitcast_convert_type(b, jnp.float32)


def _body(x_ref, o_ref, scr_ref, tm_ref):
    i = pl.program_id(0)

    @pl.when(i < _NBLK)
    def _load():
        x = x_ref[...]
        b = jax.lax.bitcast_convert_type(x, jnp.int32)
        keys = b ^ (jax.lax.shift_right_arithmetic(b, 31) & _MASK31)
        scr_ref[pl.ds(i * _ROWS_PER_BLK, _ROWS_PER_BLK), :] = keys

    @pl.when(i == _NBLK)
    def _search():
        def count_lt(q):
            def chunk(j, acc):
                c = scr_ref[pl.ds(j * _ROWS_PER_BLK, _ROWS_PER_BLK), :]
                m = (c < q).astype(jnp.int32)
                return acc + m.reshape(64, 4, 8, 128).sum(axis=0)
            acc = jax.lax.fori_loop(
                0, _NBLK, chunk, jnp.zeros((4, 8, 128), jnp.int32),
                unroll=8)
            return jnp.sum(acc)

        def step(s, p):
            bit = jnp.left_shift(np.int32(1), (31 - s).astype(jnp.int32))
            q = p + bit
            c = count_lt(q)
            return jnp.where(c <= _LOW_RANK, q, p)

        p = jax.lax.fori_loop(0, 32, step, _INT_MIN)

        def succ_chunk(j, carry):
            c_le, mn_above = carry
            c = scr_ref[pl.ds(j * _ROWS_PER_BLK, _ROWS_PER_BLK), :]
            c_le = c_le + jnp.sum((c <= p).astype(jnp.int32))
            above = jnp.where(c > p, c, _INT_MAX)
            return c_le, jnp.minimum(mn_above, jnp.min(above))

        c_le, mn_above = jax.lax.fori_loop(
            0, _NBLK, succ_chunk, (jnp.int32(0), _INT_MAX))
        p_high = jnp.where(c_le >= _LOW_RANK + 2, p, mn_above)

        v_low = _key_to_f32(p)
        v_high = _key_to_f32(p_high)
        tresh = v_low * _LOW_W + v_high * _HIGH_W
        tm_ref[0] = tresh
        tm_ref[1] = jnp.where(tresh >= v_high, v_high, v_low)

    @pl.when(i == _NBLK + 1)
    def _apply():
        tresh = tm_ref[0]
        m = tm_ref[1]

        def chunk(j, carry):
            keys = scr_ref[pl.ds(j * _ROWS_PER_BLK, _ROWS_PER_BLK), :]
            x = _key_to_f32(keys)
            o_ref[pl.ds(j * _ROWS_PER_BLK, _ROWS_PER_BLK), :] = (
                jnp.where(x > tresh, m, x))
            return carry

        jax.lax.fori_loop(0, _NBLK, chunk, jnp.int32(0))


@jax.jit
def kernel(tensor):
    return pl.pallas_call(
        _body,
        grid=(_NBLK + 2,),
        in_specs=[pl.BlockSpec(
            (_ROWS_PER_BLK, _SHAPE[1]),
            lambda i: (jnp.minimum(i, _NBLK - 1), 0))],
        out_specs=pl.BlockSpec(_SHAPE, lambda i: (0, 0)),
        out_shape=jax.ShapeDtypeStruct(_SHAPE, jnp.float32),
        scratch_shapes=[pltpu.VMEM(_SHAPE, jnp.int32),
                        pltpu.SMEM((2,), jnp.float32)],
    )(tensor)


# speedup vs baseline: 1.7877x; 1.1135x over previous
import jax
import jax.numpy as jnp
import numpy as np
from jax.experimental import pallas as pl
from jax.experimental.pallas import tpu as pltpu

_SHAPE = (128, 32768)
_N = _SHAPE[0] * _SHAPE[1]
_LOW_RANK = 4026530
_LOW_W = np.float32(0.25)
_HIGH_W = np.float32(0.75)
_MASK31 = np.int32(0x7FFFFFFF)
_INT_MIN = np.int32(-(2**31))
_INT_MAX = np.int32(2**31 - 1)

_ROWS_PER_BLK = 8
_NBLK = _SHAPE[0] // _ROWS_PER_BLK


def _key_to_f32(k):
    b = k ^ (jax.lax.shift_right_arithmetic(k, 31) & _MASK31)
    return jax.lax.bitcast_convert_type(b, jnp.float32)


def _body(x_ref, o_ref, scr_ref, s15_ref, tm_ref):
    i = pl.program_id(0)

    @pl.when(i < _NBLK)
    def _load():
        x = x_ref[...]
        b = jax.lax.bitcast_convert_type(x, jnp.int32)
        keys = b ^ (jax.lax.shift_right_arithmetic(b, 31) & _MASK31)
        scr_ref[pl.ds(i * _ROWS_PER_BLK, _ROWS_PER_BLK), :] = keys
        u15 = jax.lax.shift_right_logical(keys ^ _INT_MIN, 17)
        packed = (u15[:, :_SHAPE[1] // 2] << 16) | u15[:, _SHAPE[1] // 2:]
        s15_ref[pl.ds(i * _ROWS_PER_BLK, _ROWS_PER_BLK), :] = packed

    @pl.when(i == _NBLK)
    def _search():
        def count_lt(q):
            def chunk(j, acc):
                c = scr_ref[pl.ds(j * _ROWS_PER_BLK, _ROWS_PER_BLK), :]
                m = (c < q).astype(jnp.int32)
                return acc + m.reshape(64, 4, 8, 128).sum(axis=0)
            acc = jax.lax.fori_loop(
                0, _NBLK, chunk, jnp.zeros((4, 8, 128), jnp.int32),
                unroll=8)
            return jnp.sum(acc)

        def count_lt_swar(q):
            q15 = jax.lax.shift_right_logical(q ^ _INT_MIN, 17)
            cc = (q15 + np.int32(0x7FFF)) * np.int32(0x00010001)

            def chunk(j, acc):
                c = s15_ref[pl.ds(j * _ROWS_PER_BLK, _ROWS_PER_BLK), :]
                bits = jax.lax.shift_right_logical(cc - c, 15) \
                    & np.int32(0x00010001)
                return acc + bits.reshape(32, 4, 8, 128).sum(axis=0)
            acc = jax.lax.fori_loop(
                0, _NBLK, chunk, jnp.zeros((4, 8, 128), jnp.int32),
                unroll=8)
            return (jnp.sum(acc & np.int32(0xFFFF))
                    + jnp.sum(jax.lax.shift_right_logical(acc, 16)))

        def step_swar(s, p):
            bit = jnp.left_shift(np.int32(1), (31 - s).astype(jnp.int32))
            q = p + bit
            c = count_lt_swar(q)
            return jnp.where(c <= _LOW_RANK, q, p)

        def step(s, p):
            bit = jnp.left_shift(np.int32(1), (31 - s).astype(jnp.int32))
            q = p + bit
            c = count_lt(q)
            return jnp.where(c <= _LOW_RANK, q, p)

        p = jax.lax.fori_loop(0, 15, step_swar, _INT_MIN)
        p = jax.lax.fori_loop(15, 32, step, p)

        def succ_chunk(j, carry):
            c_le, mn_above = carry
            c = scr_ref[pl.ds(j * _ROWS_PER_BLK, _ROWS_PER_BLK), :]
            c_le = c_le + jnp.sum((c <= p).astype(jnp.int32))
            above = jnp.where(c > p, c, _INT_MAX)
            return c_le, jnp.minimum(mn_above, jnp.min(above))

        c_le, mn_above = jax.lax.fori_loop(
            0, _NBLK, succ_chunk, (jnp.int32(0), _INT_MAX))
        p_high = jnp.where(c_le >= _LOW_RANK + 2, p, mn_above)

        v_low = _key_to_f32(p)
        v_high = _key_to_f32(p_high)
        tresh = v_low * _LOW_W + v_high * _HIGH_W
        tm_ref[0] = tresh
        tm_ref[1] = jnp.where(tresh >= v_high, v_high, v_low)

    @pl.when(i == _NBLK + 1)
    def _apply():
        tresh = tm_ref[0]
        m = tm_ref[1]

        def chunk(j, carry):
            keys = scr_ref[pl.ds(j * _ROWS_PER_BLK, _ROWS_PER_BLK), :]
            x = _key_to_f32(keys)
            o_ref[pl.ds(j * _ROWS_PER_BLK, _ROWS_PER_BLK), :] = (
                jnp.where(x > tresh, m, x))
            return carry

        jax.lax.fori_loop(0, _NBLK, chunk, jnp.int32(0))


@jax.jit
def kernel(tensor):
    return pl.pallas_call(
        _body,
        grid=(_NBLK + 2,),
        in_specs=[pl.BlockSpec(
            (_ROWS_PER_BLK, _SHAPE[1]),
            lambda i: (jnp.minimum(i, _NBLK - 1), 0))],
        out_specs=pl.BlockSpec(_SHAPE, lambda i: (0, 0)),
        out_shape=jax.ShapeDtypeStruct(_SHAPE, jnp.float32),
        scratch_shapes=[pltpu.VMEM(_SHAPE, jnp.int32),
                        pltpu.VMEM((_SHAPE[0], _SHAPE[1] // 2), jnp.int32),
                        pltpu.SMEM((2,), jnp.float32)],
    )(tensor)


# speedup vs baseline: 1.9786x; 1.1068x over previous
import jax
import jax.numpy as jnp
import numpy as np
from jax.experimental import pallas as pl
from jax.experimental.pallas import tpu as pltpu

_SHAPE = (128, 32768)
_N = _SHAPE[0] * _SHAPE[1]
_LOW_RANK = 4026530
_LOW_W = np.float32(0.25)
_HIGH_W = np.float32(0.75)
_MASK31 = np.int32(0x7FFFFFFF)
_INT_MIN = np.int32(-(2**31))
_INT_MAX = np.int32(2**31 - 1)

_ROWS_PER_BLK = 8
_NBLK = _SHAPE[0] // _ROWS_PER_BLK


def _key_to_f32(k):
    b = k ^ (jax.lax.shift_right_arithmetic(k, 31) & _MASK31)
    return jax.lax.bitcast_convert_type(b, jnp.float32)


def _body(x_ref, o_ref, scr_ref, s15_ref, tm_ref):
    i = pl.program_id(0)

    @pl.when(i < _NBLK)
    def _load():
        x = x_ref[...]
        b = jax.lax.bitcast_convert_type(x, jnp.int32)
        keys = b ^ (jax.lax.shift_right_arithmetic(b, 31) & _MASK31)
        scr_ref[pl.ds(i * _ROWS_PER_BLK, _ROWS_PER_BLK), :] = keys
        u15 = jax.lax.shift_right_logical(keys ^ _INT_MIN, 17)
        packed = (u15[:, :_SHAPE[1] // 2] << 16) | u15[:, _SHAPE[1] // 2:]
        s15_ref[pl.ds(i * _ROWS_PER_BLK, _ROWS_PER_BLK), :] = packed

    @pl.when(i == _NBLK)
    def _search():
        def count_lt(q):
            def chunk(j, acc):
                c = scr_ref[pl.ds(j * _ROWS_PER_BLK, _ROWS_PER_BLK), :]
                m = (c < q).astype(jnp.int32)
                return acc + m.reshape(64, 4, 8, 128).sum(axis=0)
            acc = jax.lax.fori_loop(
                0, _NBLK, chunk, jnp.zeros((4, 8, 128), jnp.int32),
                unroll=8)
            return jnp.sum(acc)

        def count_swar(q15):
            cc = (q15 + np.int32(0x7FFF)) * np.int32(0x00010001)

            def chunk(j, acc):
                c = s15_ref[pl.ds(j * _ROWS_PER_BLK, _ROWS_PER_BLK), :]
                bits = jax.lax.shift_right_logical(cc - c, 15) \
                    & np.int32(0x00010001)
                return acc + bits.reshape(32, 4, 8, 128).sum(axis=0)
            acc = jax.lax.fori_loop(
                0, _NBLK, chunk, jnp.zeros((4, 8, 128), jnp.int32),
                unroll=8)
            return (jnp.sum(acc & np.int32(0xFFFF))
                    + jnp.sum(jax.lax.shift_right_logical(acc, 16)))

        def step_swar(s, carry):
            p, c_lo = carry
            bit = jnp.left_shift(np.int32(1), (31 - s).astype(jnp.int32))
            q = p + bit
            c = count_swar(jax.lax.shift_right_logical(q ^ _INT_MIN, 17))
            take = c <= _LOW_RANK
            return jnp.where(take, q, p), jnp.where(take, c, c_lo)

        def step(s, carry):
            p, c_lo = carry
            bit = jnp.left_shift(np.int32(1), (31 - s).astype(jnp.int32))
            q = p + bit
            c = count_lt(q)
            take = c <= _LOW_RANK
            return jnp.where(take, q, p), jnp.where(take, c, c_lo)

        p, c_lo = jax.lax.fori_loop(0, 15, step_swar,
                                    (_INT_MIN, jnp.int32(0)))
        p, c_lo = jax.lax.fori_loop(15, 17, step, (p, c_lo))

        def repack(j, carry):
            k = scr_ref[pl.ds(j * _ROWS_PER_BLK, _ROWS_PER_BLK), :]
            inb = jax.lax.shift_right_logical(k ^ p, 15) == 0
            m15 = jnp.where(inb, k & np.int32(0x7FFF), np.int32(0x7FFF))
            s15_ref[pl.ds(j * _ROWS_PER_BLK, _ROWS_PER_BLK), :] = (
                (m15[:, :_SHAPE[1] // 2] << 16) | m15[:, _SHAPE[1] // 2:])
            return carry

        jax.lax.fori_loop(0, _NBLK, repack, jnp.int32(0))

        def step_low(s, carry):
            p, c_lo = carry
            bit = jnp.left_shift(np.int32(1), (14 - s).astype(jnp.int32))
            q = p + bit
            c = c_lo_base + count_swar(q & np.int32(0x7FFF))
            take = c <= _LOW_RANK
            return jnp.where(take, q, p), jnp.where(take, c, c_lo)

        c_lo_base = c_lo
        p, c_lo = jax.lax.fori_loop(0, 15, step_low, (p, c_lo))

        def succ_chunk(j, carry):
            c_le, mn_above = carry
            c = scr_ref[pl.ds(j * _ROWS_PER_BLK, _ROWS_PER_BLK), :]
            c_le = c_le + jnp.sum((c <= p).astype(jnp.int32))
            above = jnp.where(c > p, c, _INT_MAX)
            return c_le, jnp.minimum(mn_above, jnp.min(above))

        c_le, mn_above = jax.lax.fori_loop(
            0, _NBLK, succ_chunk, (jnp.int32(0), _INT_MAX))
        p_high = jnp.where(c_le >= _LOW_RANK + 2, p, mn_above)

        v_low = _key_to_f32(p)
        v_high = _key_to_f32(p_high)
        tresh = v_low * _LOW_W + v_high * _HIGH_W
        tm_ref[0] = tresh
        tm_ref[1] = jnp.where(tresh >= v_high, v_high, v_low)

    @pl.when(i == _NBLK + 1)
    def _apply():
        tresh = tm_ref[0]
        m = tm_ref[1]

        def chunk(j, carry):
            keys = scr_ref[pl.ds(j * _ROWS_PER_BLK, _ROWS_PER_BLK), :]
            x = _key_to_f32(keys)
            o_ref[pl.ds(j * _ROWS_PER_BLK, _ROWS_PER_BLK), :] = (
                jnp.where(x > tresh, m, x))
            return carry

        jax.lax.fori_loop(0, _NBLK, chunk, jnp.int32(0))


@jax.jit
def kernel(tensor):
    return pl.pallas_call(
        _body,
        grid=(_NBLK + 2,),
        in_specs=[pl.BlockSpec(
            (_ROWS_PER_BLK, _SHAPE[1]),
            lambda i: (jnp.minimum(i, _NBLK - 1), 0))],
        out_specs=pl.BlockSpec(_SHAPE, lambda i: (0, 0)),
        out_shape=jax.ShapeDtypeStruct(_SHAPE, jnp.float32),
        scratch_shapes=[pltpu.VMEM(_SHAPE, jnp.int32),
                        pltpu.VMEM((_SHAPE[0], _SHAPE[1] // 2), jnp.int32),
                        pltpu.SMEM((2,), jnp.float32)],
    )(tensor)


# speedup vs baseline: 2.0303x; 1.0261x over previous
import jax
import jax.numpy as jnp
import numpy as np
from jax.experimental import pallas as pl
from jax.experimental.pallas import tpu as pltpu

_SHAPE = (128, 32768)
_N = _SHAPE[0] * _SHAPE[1]
_LOW_RANK = 4026530
_LOW_W = np.float32(0.25)
_HIGH_W = np.float32(0.75)
_MASK31 = np.int32(0x7FFFFFFF)
_INT_MIN = np.int32(-(2**31))
_INT_MAX = np.int32(2**31 - 1)

_ROWS_PER_BLK = 8
_NBLK = _SHAPE[0] // _ROWS_PER_BLK


def _key_to_f32(k):
    b = k ^ (jax.lax.shift_right_arithmetic(k, 31) & _MASK31)
    return jax.lax.bitcast_convert_type(b, jnp.float32)


def _body(x_ref, o_ref, scr_ref, s15_ref, tm_ref):
    i = pl.program_id(0)

    @pl.when(i < _NBLK)
    def _load():
        x = x_ref[...]
        b = jax.lax.bitcast_convert_type(x, jnp.int32)
        keys = b ^ (jax.lax.shift_right_arithmetic(b, 31) & _MASK31)
        scr_ref[pl.ds(i * _ROWS_PER_BLK, _ROWS_PER_BLK), :] = keys
        u15 = jax.lax.shift_right_logical(keys ^ _INT_MIN, 17)
        packed = (u15[:, :_SHAPE[1] // 2] << 16) | u15[:, _SHAPE[1] // 2:]
        s15_ref[pl.ds(i * _ROWS_PER_BLK, _ROWS_PER_BLK), :] = packed

    @pl.when(i == _NBLK)
    def _search():
        def count_lt(q):
            def chunk(j, acc):
                c = scr_ref[pl.ds(j * _ROWS_PER_BLK, _ROWS_PER_BLK), :]
                m = (c < q).astype(jnp.int32)
                return acc + m.reshape(64, 4, 8, 128).sum(axis=0)
            acc = jax.lax.fori_loop(
                0, _NBLK, chunk, jnp.zeros((4, 8, 128), jnp.int32),
                unroll=8)
            return jnp.sum(acc)

        def count_swar(q15):
            cc = (q15 + np.int32(0x7FFF)) * np.int32(0x00010001)

            def chunk(j, acc):
                c = s15_ref[pl.ds(j * _ROWS_PER_BLK, _ROWS_PER_BLK), :]
                bits = jax.lax.shift_right_logical(cc - c, 15) \
                    & np.int32(0x00010001)
                return acc + bits.reshape(32, 4, 8, 128).sum(axis=0)
            acc = jax.lax.fori_loop(
                0, _NBLK, chunk, jnp.zeros((4, 8, 128), jnp.int32),
                unroll=8)
            return (jnp.sum(acc & np.int32(0xFFFF))
                    + jnp.sum(jax.lax.shift_right_logical(acc, 16)))

        def step_swar(s, carry):
            p, c_lo = carry
            bit = jnp.left_shift(np.int32(1), (31 - s).astype(jnp.int32))
            q = p + bit
            c = count_swar(jax.lax.shift_right_logical(q ^ _INT_MIN, 17))
            take = c <= _LOW_RANK
            return jnp.where(take, q, p), jnp.where(take, c, c_lo)

        def step(s, carry):
            p, c_lo = carry
            bit = jnp.left_shift(np.int32(1), (31 - s).astype(jnp.int32))
            q = p + bit
            c = count_lt(q)
            take = c <= _LOW_RANK
            return jnp.where(take, q, p), jnp.where(take, c, c_lo)

        p, c_lo = jax.lax.fori_loop(0, 15, step_swar,
                                    (_INT_MIN, jnp.int32(0)))
        p, c_lo = jax.lax.fori_loop(15, 17, step, (p, c_lo))

        def repack(j, carry):
            k = scr_ref[pl.ds(j * _ROWS_PER_BLK, _ROWS_PER_BLK), :]
            inb = jax.lax.shift_right_logical(k ^ p, 15) == 0
            m15 = jnp.where(inb, k & np.int32(0x7FFF), np.int32(0x7FFF))
            s15_ref[pl.ds(j * _ROWS_PER_BLK, _ROWS_PER_BLK), :] = (
                (m15[:, :_SHAPE[1] // 2] << 16) | m15[:, _SHAPE[1] // 2:])
            return carry

        jax.lax.fori_loop(0, _NBLK, repack, jnp.int32(0), unroll=4)

        def step_low(s, carry):
            p, c_lo = carry
            bit = jnp.left_shift(np.int32(1), (14 - s).astype(jnp.int32))
            q = p + bit
            c = c_lo_base + count_swar(q & np.int32(0x7FFF))
            take = c <= _LOW_RANK
            return jnp.where(take, q, p), jnp.where(take, c, c_lo)

        c_lo_base = c_lo
        p, c_lo = jax.lax.fori_loop(0, 15, step_low, (p, c_lo))

        def succ_chunk(j, carry):
            c_le, mn_above = carry
            c = scr_ref[pl.ds(j * _ROWS_PER_BLK, _ROWS_PER_BLK), :]
            c_le = c_le + jnp.sum((c <= p).astype(jnp.int32))
            above = jnp.where(c > p, c, _INT_MAX)
            return c_le, jnp.minimum(mn_above, jnp.min(above))

        c_le, mn_above = jax.lax.fori_loop(
            0, _NBLK, succ_chunk, (jnp.int32(0), _INT_MAX), unroll=4)
        p_high = jnp.where(c_le >= _LOW_RANK + 2, p, mn_above)

        v_low = _key_to_f32(p)
        v_high = _key_to_f32(p_high)
        tresh = v_low * _LOW_W + v_high * _HIGH_W
        tm_ref[0] = tresh
        tm_ref[1] = jnp.where(tresh >= v_high, v_high, v_low)

    @pl.when(i == _NBLK + 1)
    def _apply():
        tresh = tm_ref[0]
        m = tm_ref[1]

        def chunk(j, carry):
            keys = scr_ref[pl.ds(j * _ROWS_PER_BLK, _ROWS_PER_BLK), :]
            x = _key_to_f32(keys)
            o_ref[pl.ds(j * _ROWS_PER_BLK, _ROWS_PER_BLK), :] = (
                jnp.where(x > tresh, m, x))
            return carry

        jax.lax.fori_loop(0, _NBLK, chunk, jnp.int32(0), unroll=4)


@jax.jit
def kernel(tensor):
    return pl.pallas_call(
        _body,
        grid=(_NBLK + 2,),
        in_specs=[pl.BlockSpec(
            (_ROWS_PER_BLK, _SHAPE[1]),
            lambda i: (jnp.minimum(i, _NBLK - 1), 0))],
        out_specs=pl.BlockSpec(_SHAPE, lambda i: (0, 0)),
        out_shape=jax.ShapeDtypeStruct(_SHAPE, jnp.float32),
        scratch_shapes=[pltpu.VMEM(_SHAPE, jnp.int32),
                        pltpu.VMEM((_SHAPE[0], _SHAPE[1] // 2), jnp.int32),
                        pltpu.SMEM((2,), jnp.float32)],
    )(tensor)
